# baseline (speedup 1.0000x reference)
import jax
import jax.numpy as jnp
from jax.experimental import pallas as pl
from jax.experimental.pallas import tpu as pltpu

NUM_SEGMENTS = 64
ROW_BLOCK = 2000
COL_BLOCK = 256


def _reduce_body(pl_ref, pr_ref, t_ref, o_ref):
    i = pl.program_id(0)

    @pl.when(i == 0)
    def _init():
        o_ref[0, 0] = 0.0

    s = (jnp.sum(jnp.abs(pl_ref[...] - t_ref[:, :COL_BLOCK]))
         + jnp.sum(jnp.abs(pr_ref[...] - t_ref[:, COL_BLOCK:])))
    o_ref[0, 0] += s

    @pl.when(i == pl.num_programs(0) - 1)
    def _finalize():
        o_ref[0, 0] = o_ref[0, 0] / (NUM_SEGMENTS * 512.0)


def kernel(preds, target, batch_map):
    n_rows, n_cols = preds.shape
    grid = (n_rows // ROW_BLOCK,)
    out = pl.pallas_call(
        _reduce_body,
        grid=grid,
        in_specs=[
            pl.BlockSpec((ROW_BLOCK, COL_BLOCK), lambda i: (i, 0)),
            pl.BlockSpec((ROW_BLOCK, COL_BLOCK), lambda i: (i, 1)),
            pl.BlockSpec((ROW_BLOCK, n_cols), lambda i: (i, 0)),
        ],
        out_specs=pl.BlockSpec(
            (1, 1), lambda i: (0, 0), memory_space=pltpu.SMEM
        ),
        out_shape=jax.ShapeDtypeStruct((1, 1), jnp.float32),
        compiler_params=pltpu.CompilerParams(
            dimension_semantics=("arbitrary",),
        ),
    )(preds, preds, target)
    return out[0, 0]
